# baseline (speedup 1.0000x reference)
import functools

import jax
import jax.numpy as jnp
from jax import lax
from jax.experimental import pallas as pl
from jax.experimental.pallas import tpu as pltpu
from jax.experimental.pallas import tpu_sc as plsc

D_MODEL = 128
CHUNK = 128
NBUF = 5
LAG = 2


def _body(idx_hbm, table_hbm, out_hbm, idx_v, rows_v, *sems):
    gsems = list(sems[:NBUF])
    wsems = list(sems[NBUF:])
    info = plsc.get_sparse_core_info()
    nc = info.num_cores
    wid = lax.axis_index("s") * nc + lax.axis_index("c")
    n_ch = idx_v.shape[0]
    b0 = wid * CHUNK

    def g_start(j, b):
        pltpu.async_copy(table_hbm.at[idx_v.at[j]], rows_v.at[b], gsems[b])

    def g_wait(j, b):
        pltpu.make_async_copy(
            table_hbm.at[idx_v.at[j]], rows_v.at[b], gsems[b]
        ).wait()

    def add1(r):
        for c in range(CHUNK // 16):
            sl = pl.ds(c * 16, 16)
            idx_v[r, sl] = idx_v[r, sl] + 1

    def w_start(j, b):
        pltpu.async_copy(
            rows_v.at[b], out_hbm.at[j, pl.ds(b0, CHUNK)], wsems[b]
        )

    def w_wait(j, b):
        pltpu.make_async_copy(
            rows_v.at[b], out_hbm.at[j, pl.ds(b0, CHUNK)], wsems[b]
        ).wait()

    pltpu.sync_copy(idx_hbm.at[:, pl.ds(b0, CHUNK)], idx_v)

    for j in range(LAG):
        add1(j)
        g_start(j, j % NBUF)

    def _step(g, _):
        for b in range(NBUF):
            j = g * NBUF + b
            g_wait(j, b)
            w_start(j, b)

            @pl.when(j >= NBUF - LAG)
            def _():
                w_wait(j - (NBUF - LAG), (b + LAG) % NBUF)

            @pl.when(j + LAG < n_ch)
            def _():
                add1(j + LAG)
                g_start(j + LAG, (b + LAG) % NBUF)

        return 0

    lax.fori_loop(0, n_ch // NBUF, _step, 0)

    for j in range(n_ch - (NBUF - LAG), n_ch):
        w_wait(j, j % NBUF)


def kernel(x, parent_embeddings):
    batch, seq = x.shape
    idx = x.T.astype(jnp.int32)

    mesh = plsc.VectorSubcoreMesh(core_axis_name="c", subcore_axis_name="s")
    run = functools.partial(
        pl.kernel,
        mesh=mesh,
        out_type=jax.ShapeDtypeStruct((seq, batch, D_MODEL), jnp.float32),
        scratch_types=[
            pltpu.VMEM((seq, CHUNK), jnp.int32),
            pltpu.VMEM((NBUF, CHUNK, D_MODEL), jnp.float32),
        ]
        + [pltpu.SemaphoreType.DMA] * (2 * NBUF),
    )(_body)
    out = run(idx, parent_embeddings)
    return jnp.transpose(out, (1, 0, 2))


# speedup vs baseline: 1.0109x; 1.0109x over previous
import functools

import jax
import jax.numpy as jnp
from jax import lax
from jax.experimental import pallas as pl
from jax.experimental.pallas import tpu as pltpu
from jax.experimental.pallas import tpu_sc as plsc

D_MODEL = 128
CHUNK = 128
NBUF = 5
LAG = 3


def _body(idx_hbm, table_hbm, out_hbm, idx_v, rows_v, *sems):
    gsems = list(sems[:NBUF])
    wsems = list(sems[NBUF:])
    info = plsc.get_sparse_core_info()
    nc = info.num_cores
    wid = lax.axis_index("s") * nc + lax.axis_index("c")
    n_ch = idx_v.shape[0]
    b0 = wid * CHUNK

    def g_start(j, b):
        pltpu.async_copy(table_hbm.at[idx_v.at[j]], rows_v.at[b], gsems[b])

    def g_wait(j, b):
        pltpu.make_async_copy(
            table_hbm.at[idx_v.at[j]], rows_v.at[b], gsems[b]
        ).wait()

    def add1(r):
        for c in range(CHUNK // 16):
            sl = pl.ds(c * 16, 16)
            idx_v[r, sl] = idx_v[r, sl] + 1

    def w_start(j, b):
        pltpu.async_copy(
            rows_v.at[b], out_hbm.at[j, pl.ds(b0, CHUNK)], wsems[b]
        )

    def w_wait(j, b):
        pltpu.make_async_copy(
            rows_v.at[b], out_hbm.at[j, pl.ds(b0, CHUNK)], wsems[b]
        ).wait()

    pltpu.sync_copy(idx_hbm.at[:, pl.ds(b0, CHUNK)], idx_v)

    for j in range(LAG):
        add1(j)
        g_start(j, j % NBUF)

    def _step(g, _):
        for b in range(NBUF):
            j = g * NBUF + b
            g_wait(j, b)
            w_start(j, b)

            @pl.when(j >= NBUF - LAG)
            def _():
                w_wait(j - (NBUF - LAG), (b + LAG) % NBUF)

            @pl.when(j + LAG < n_ch)
            def _():
                add1(j + LAG)
                g_start(j + LAG, (b + LAG) % NBUF)

        return 0

    lax.fori_loop(0, n_ch // NBUF, _step, 0)

    for j in range(n_ch - (NBUF - LAG), n_ch):
        w_wait(j, j % NBUF)


def kernel(x, parent_embeddings):
    batch, seq = x.shape
    idx = x.T.astype(jnp.int32)

    mesh = plsc.VectorSubcoreMesh(core_axis_name="c", subcore_axis_name="s")
    run = functools.partial(
        pl.kernel,
        mesh=mesh,
        out_type=jax.ShapeDtypeStruct((seq, batch, D_MODEL), jnp.float32),
        scratch_types=[
            pltpu.VMEM((seq, CHUNK), jnp.int32),
            pltpu.VMEM((NBUF, CHUNK, D_MODEL), jnp.float32),
        ]
        + [pltpu.SemaphoreType.DMA] * (2 * NBUF),
    )(_body)
    out = run(idx, parent_embeddings)
    return jnp.transpose(out, (1, 0, 2))
